# baseline (speedup 1.0000x reference)
import functools

import jax
import jax.numpy as jnp
from jax import lax
from jax.experimental import pallas as pl
from jax.experimental.pallas import tpu as pltpu
from jax.experimental.pallas import tpu_sc as plsc

N = 10000
D = 128
E = 320000

NUM_CORES = 2
NUM_SUBCORES = 16
NUM_WORKERS = NUM_CORES * NUM_SUBCORES

CHUNK = 64
NBUF = 5
LEAD = 3
GROUP = 4
NIG = 4
SUPER = 20
CHUNKS_PER_TILE = 160
NUM_GROUPS = CHUNKS_PER_TILE // GROUP
NUM_SUPER = CHUNKS_PER_TILE // SUPER
E_PAD = NUM_WORKERS * CHUNKS_PER_TILE * CHUNK

N_PAD = 10240
ROWS_PER_TILE = N_PAD // NUM_SUBCORES

def _fill_ones(ones_v):
    for i in range(CHUNK // 16):
        ones_v[pl.ds(i * 16, 16)] = jnp.ones((16,), jnp.float32)


DEG_SEMS = 8


def _deg_body(dst_hbm, out_hbm, dst_v, ones_v, hist_s, sems):
    c = lax.axis_index("c")
    s = lax.axis_index("s")
    wid = s * NUM_CORES + c

    _fill_ones(ones_v)
    for k in range(ROWS_PER_TILE // CHUNK):
        pltpu.sync_copy(ones_v, hist_s.at[pl.ds(s * ROWS_PER_TILE + k * CHUNK, CHUNK)])
    pltpu.sync_copy(dst_hbm.at[wid], dst_v)
    plsc.subcore_barrier()

    def _add(j, t):
        return pltpu.make_async_copy(
            ones_v, hist_s.at[dst_v.at[j]], sems.at[t])

    for t in range(DEG_SEMS):
        pltpu.async_copy(ones_v, hist_s.at[dst_v.at[t]], sems.at[t], add=True)

    @pl.loop(1, CHUNKS_PER_TILE // DEG_SEMS)
    def _(blk):
        j0 = blk * DEG_SEMS
        for t in range(DEG_SEMS):
            _add(j0 + t, t).wait()
            pltpu.async_copy(ones_v, hist_s.at[dst_v.at[j0 + t]],
                             sems.at[t], add=True)

    for t in range(DEG_SEMS):
        _add(0, t).wait()

    plsc.subcore_barrier()
    stripe = pl.ds(s * ROWS_PER_TILE, ROWS_PER_TILE)
    pltpu.sync_copy(hist_s.at[stripe], out_hbm.at[c, stripe])


def _scatter_body(g_hbm, edges_hbm, out_hbm, rows_v, iring_v, acc_s, rsems,
                  isems, ssems):
    c = lax.axis_index("c")
    s = lax.axis_index("s")
    wid = s * NUM_CORES + c
    stripe = pl.ds(s * ROWS_PER_TILE, ROWS_PER_TILE)

    def _idx_load(grp, slot):
        return pltpu.make_async_copy(
            edges_hbm.at[wid, grp], iring_v.at[slot], isems.at[slot])

    def _gather(gslot, row, b):
        return pltpu.make_async_copy(
            g_hbm.at[iring_v.at[gslot, row]], rows_v.at[b], rsems.at[b])

    def _scatter_drain(b):
        pltpu.make_async_copy(
            g_hbm.at[pl.ds(0, CHUNK)], rows_v.at[b], ssems.at[b]).wait()

    def _slot(ss, t, edge_ss):
        static = edge_ss is not None
        rem = (lambda a, m: a % m) if static else lax.rem
        q0 = (SUPER // GROUP) * ss
        b, k = t % NBUF, t % GROUP
        q = q0 + t // GROUP
        if k == 0 and not (static and edge_ss == NUM_SUPER - 1 and t >= 12):
            _idx_load(q + 2, rem(q + 2, NIG)).start()
        _gather(rem(q, NIG), 2 * k, b).wait()
        pltpu.async_copy(
            rows_v.at[b], acc_s.at[iring_v.at[rem(q, NIG), 2 * k + 1]],
            ssems.at[b], add=True)
        if static and edge_ss == NUM_SUPER - 1 and t >= SUPER - LEAD:
            return
        t2 = t + LEAD
        q2, k2, b2 = q0 + t2 // GROUP, t2 % GROUP, t2 % NBUF
        if k2 == 0:
            _idx_load(q2, rem(q2, NIG)).wait()
        if not (static and edge_ss == 0 and t + LEAD < NBUF):
            _scatter_drain(b2)
        _gather(rem(q2, NIG), 2 * k2, b2).start()

    @pl.when(c == 0)
    def _():
        pltpu.sync_copy(g_hbm.at[stripe], acc_s.at[stripe])

    @pl.when(c == 1)
    def _():
        for r in range(CHUNK):
            for i in range(D // 16):
                rows_v[0, r, pl.ds(16 * i, 16)] = jnp.zeros((16,), jnp.float32)
        for m in range(ROWS_PER_TILE // CHUNK):
            pltpu.sync_copy(
                rows_v.at[0],
                acc_s.at[pl.ds(s * ROWS_PER_TILE + m * CHUNK, CHUNK)])

    _idx_load(0, 0).start()
    _idx_load(1, 1).start()
    plsc.subcore_barrier()
    _idx_load(0, 0).wait()
    for j in range(LEAD):
        _gather(0, 2 * j, j).start()

    for t in range(SUPER):
        _slot(0, t, 0)

    @pl.loop(1, NUM_SUPER - 1)
    def _(ss):
        for t in range(SUPER):
            _slot(ss, t, None)

    for t in range(SUPER):
        _slot(NUM_SUPER - 1, t, NUM_SUPER - 1)

    for j in range(CHUNKS_PER_TILE - NBUF, CHUNKS_PER_TILE):
        _scatter_drain(j % NBUF)

    plsc.subcore_barrier()
    pltpu.sync_copy(acc_s.at[stripe], out_hbm.at[c, stripe])


def _dinv(deg_ref):
    return lax.rsqrt(deg_ref[:, 0:1] + deg_ref[:, 1:2] - 1.0)


def _tc1_body(deg_ref, x_ref, w_ref, g_ref):
    h = jnp.dot(x_ref[...], w_ref[...], preferred_element_type=jnp.float32)
    g_ref[0:N] = h * _dinv(deg_ref)[0:N]
    g_ref[pl.ds(N, N_PAD - N)] = jnp.zeros((N_PAD - N, D), jnp.float32)


def _tc2_body(deg_ref, s_ref, w_ref, b_ref, g2_ref):
    dinv = _dinv(deg_ref)
    z1 = jnp.maximum((s_ref[0] + s_ref[1]) * dinv + b_ref[...], 0.0)
    h2 = jnp.dot(z1, w_ref[...], preferred_element_type=jnp.float32)
    g2_ref[...] = h2 * dinv


def _tc3_body(deg_ref, s_ref, b_ref, z_ref):
    z_ref[...] = ((s_ref[0, 0:N] + s_ref[1, 0:N]) * _dinv(deg_ref)[0:N]
                  + b_ref[...])


_f32 = jnp.float32
_tc1 = pl.pallas_call(_tc1_body, out_shape=jax.ShapeDtypeStruct((N_PAD, D), _f32))
_tc2 = pl.pallas_call(_tc2_body, out_shape=jax.ShapeDtypeStruct((N_PAD, D), _f32))
_tc3 = pl.pallas_call(_tc3_body, out_shape=jax.ShapeDtypeStruct((N, D), _f32))


@functools.lru_cache(maxsize=None)
def _sc_kernels():
    mesh = plsc.VectorSubcoreMesh(
        core_axis_name="c", subcore_axis_name="s",
        num_cores=NUM_CORES, num_subcores=NUM_SUBCORES)
    deg = pl.kernel(
        _deg_body,
        mesh=mesh,
        out_type=jax.ShapeDtypeStruct((NUM_CORES, N_PAD), jnp.float32),
        scratch_types=[
            pltpu.VMEM((CHUNKS_PER_TILE, CHUNK), jnp.int32),
            pltpu.VMEM((CHUNK,), jnp.float32),
            pltpu.VMEM_SHARED((N_PAD,), jnp.float32),
            pltpu.SemaphoreType.DMA((DEG_SEMS,)),
        ],
    )
    scatter = pl.kernel(
        _scatter_body,
        mesh=mesh,
        out_type=jax.ShapeDtypeStruct((NUM_CORES, N_PAD, D), jnp.float32),
        scratch_types=[
            pltpu.VMEM((NBUF, CHUNK, D), jnp.float32),
            pltpu.VMEM((NIG, 2 * GROUP, CHUNK), jnp.int32),
            pltpu.VMEM_SHARED((N_PAD, D), jnp.float32),
            pltpu.SemaphoreType.DMA((NBUF,)),
            pltpu.SemaphoreType.DMA((NIG,)),
            pltpu.SemaphoreType.DMA((NBUF,)),
        ],
    )
    return deg, scatter


@jax.jit
def kernel(x, edge_index, W1, b1, W2, b2):
    pad = E_PAD - E
    ar = jnp.arange(pad, dtype=jnp.int32)
    src_p = jnp.concatenate([edge_index[0], ar % N])
    dst_p = jnp.concatenate([edge_index[1], N + (ar % (N_PAD - N))])
    src_p = src_p.reshape(NUM_WORKERS, CHUNKS_PER_TILE, CHUNK)
    dst_p = dst_p.reshape(NUM_WORKERS, CHUNKS_PER_TILE, CHUNK)
    edges = jnp.stack([src_p, dst_p], axis=2).reshape(
        NUM_WORKERS, NUM_GROUPS, 2 * GROUP, CHUNK)

    b1r = b1.reshape(1, D)
    b2r = b2.reshape(1, D)

    deg_kernel, scatter_kernel = _sc_kernels()
    deg_t = deg_kernel(dst_p).T
    g1 = _tc1(deg_t, x, W1)
    s1 = scatter_kernel(g1, edges)
    g2 = _tc2(deg_t, s1, W2, b1r)
    s2 = scatter_kernel(g2, edges)
    return _tc3(deg_t, s2, b2r)


# speedup vs baseline: 1.0100x; 1.0100x over previous
import functools

import jax
import jax.numpy as jnp
from jax import lax
from jax.experimental import pallas as pl
from jax.experimental.pallas import tpu as pltpu
from jax.experimental.pallas import tpu_sc as plsc

N = 10000
D = 128
E = 320000

NUM_CORES = 2
NUM_SUBCORES = 16
NUM_WORKERS = NUM_CORES * NUM_SUBCORES

CHUNK = 64
NBUF = 5
LEAD = 3
GROUP = 4
NIG = 4
SUPER = 20
CHUNKS_PER_TILE = 160
NUM_GROUPS = CHUNKS_PER_TILE // GROUP
NUM_SUPER = CHUNKS_PER_TILE // SUPER
E_PAD = NUM_WORKERS * CHUNKS_PER_TILE * CHUNK

N_PAD = 10240
ROWS_PER_TILE = N_PAD // NUM_SUBCORES

def _fill_ones(ones_v):
    for i in range(CHUNK // 16):
        ones_v[pl.ds(i * 16, 16)] = jnp.ones((16,), jnp.float32)


DEG_SEMS = 8


def _deg_body(dst_hbm, out_hbm, dst_v, ones_v, hist_s, sems):
    c = lax.axis_index("c")
    s = lax.axis_index("s")
    wid = s * NUM_CORES + c

    _fill_ones(ones_v)
    for k in range(ROWS_PER_TILE // CHUNK):
        pltpu.sync_copy(ones_v, hist_s.at[pl.ds(s * ROWS_PER_TILE + k * CHUNK, CHUNK)])
    pltpu.sync_copy(dst_hbm.at[wid], dst_v)
    plsc.subcore_barrier()

    def _add(j, t):
        return pltpu.make_async_copy(
            ones_v, hist_s.at[dst_v.at[j]], sems.at[t])

    for t in range(DEG_SEMS):
        pltpu.async_copy(ones_v, hist_s.at[dst_v.at[t]], sems.at[t], add=True)

    @pl.loop(1, CHUNKS_PER_TILE // DEG_SEMS)
    def _(blk):
        j0 = blk * DEG_SEMS
        for t in range(DEG_SEMS):
            _add(j0 + t, t).wait()
            pltpu.async_copy(ones_v, hist_s.at[dst_v.at[j0 + t]],
                             sems.at[t], add=True)

    for t in range(DEG_SEMS):
        _add(0, t).wait()

    plsc.subcore_barrier()
    stripe = pl.ds(s * ROWS_PER_TILE, ROWS_PER_TILE)
    pltpu.sync_copy(hist_s.at[stripe], out_hbm.at[c, stripe])


def _scatter_body(g_hbm, edges_hbm, out_hbm, rows_v, iring_v, acc_s, rsems,
                  isems, ssems, initsem):
    c = lax.axis_index("c")
    s = lax.axis_index("s")
    wid = s * NUM_CORES + c
    stripe = pl.ds(s * ROWS_PER_TILE, ROWS_PER_TILE)

    def _idx_load(grp, slot):
        return pltpu.make_async_copy(
            edges_hbm.at[wid, grp], iring_v.at[slot], isems.at[slot])

    def _gather(gslot, row, b):
        return pltpu.make_async_copy(
            g_hbm.at[iring_v.at[gslot, row]], rows_v.at[b], rsems.at[b])

    def _scatter_drain(b):
        pltpu.make_async_copy(
            g_hbm.at[pl.ds(0, CHUNK)], rows_v.at[b], ssems.at[b]).wait()

    def _slot(ss, t, edge_ss):
        static = edge_ss is not None
        rem = (lambda a, m: a % m) if static else lax.rem
        q0 = (SUPER // GROUP) * ss
        b, k = t % NBUF, t % GROUP
        q = q0 + t // GROUP
        if k == 0 and not (static and edge_ss == NUM_SUPER - 1 and t >= 12):
            _idx_load(q + 2, rem(q + 2, NIG)).start()
        _gather(rem(q, NIG), 2 * k, b).wait()
        pltpu.async_copy(
            rows_v.at[b], acc_s.at[iring_v.at[rem(q, NIG), 2 * k + 1]],
            ssems.at[b], add=True)
        if static and edge_ss == NUM_SUPER - 1 and t >= SUPER - LEAD:
            return
        t2 = t + LEAD
        q2, k2, b2 = q0 + t2 // GROUP, t2 % GROUP, t2 % NBUF
        if k2 == 0:
            _idx_load(q2, rem(q2, NIG)).wait()
        if not (static and edge_ss == 0 and t + LEAD < NBUF):
            _scatter_drain(b2)
        _gather(rem(q2, NIG), 2 * k2, b2).start()

    _idx_load(0, 0).start()
    _idx_load(1, 1).start()

    @pl.when(c == 0)
    def _():
        pltpu.async_copy(g_hbm.at[stripe], acc_s.at[stripe], initsem)

    @pl.when(c == 1)
    def _():
        for r in range(CHUNK):
            for i in range(D // 16):
                rows_v[0, r, pl.ds(16 * i, 16)] = jnp.zeros((16,), jnp.float32)
        for m in range(ROWS_PER_TILE // CHUNK):
            pltpu.sync_copy(
                rows_v.at[0],
                acc_s.at[pl.ds(s * ROWS_PER_TILE + m * CHUNK, CHUNK)])

    _idx_load(0, 0).wait()
    for j in range(LEAD):
        _gather(0, 2 * j, j).start()

    @pl.when(c == 0)
    def _():
        pltpu.make_async_copy(g_hbm.at[stripe], acc_s.at[stripe],
                              initsem).wait()

    plsc.subcore_barrier()

    for t in range(SUPER):
        _slot(0, t, 0)

    @pl.loop(1, NUM_SUPER - 1)
    def _(ss):
        for t in range(SUPER):
            _slot(ss, t, None)

    for t in range(SUPER):
        _slot(NUM_SUPER - 1, t, NUM_SUPER - 1)

    for j in range(CHUNKS_PER_TILE - NBUF, CHUNKS_PER_TILE):
        _scatter_drain(j % NBUF)

    plsc.subcore_barrier()
    pltpu.sync_copy(acc_s.at[stripe], out_hbm.at[c, stripe])


def _dinv(deg_ref):
    return lax.rsqrt(deg_ref[:, 0:1] + deg_ref[:, 1:2] - 1.0)


def _tc1_body(deg_ref, x_ref, w_ref, g_ref):
    h = jnp.dot(x_ref[...], w_ref[...], preferred_element_type=jnp.float32)
    g_ref[0:N] = h * _dinv(deg_ref)[0:N]
    g_ref[pl.ds(N, N_PAD - N)] = jnp.zeros((N_PAD - N, D), jnp.float32)


def _tc2_body(deg_ref, s_ref, w_ref, b_ref, g2_ref):
    dinv = _dinv(deg_ref)
    z1 = jnp.maximum((s_ref[0] + s_ref[1]) * dinv + b_ref[...], 0.0)
    h2 = jnp.dot(z1, w_ref[...], preferred_element_type=jnp.float32)
    g2_ref[...] = h2 * dinv


def _tc3_body(deg_ref, s_ref, b_ref, z_ref):
    z_ref[...] = ((s_ref[0, 0:N] + s_ref[1, 0:N]) * _dinv(deg_ref)[0:N]
                  + b_ref[...])


_f32 = jnp.float32
_tc1 = pl.pallas_call(_tc1_body, out_shape=jax.ShapeDtypeStruct((N_PAD, D), _f32))
_tc2 = pl.pallas_call(_tc2_body, out_shape=jax.ShapeDtypeStruct((N_PAD, D), _f32))
_tc3 = pl.pallas_call(_tc3_body, out_shape=jax.ShapeDtypeStruct((N, D), _f32))


@functools.lru_cache(maxsize=None)
def _sc_kernels():
    mesh = plsc.VectorSubcoreMesh(
        core_axis_name="c", subcore_axis_name="s",
        num_cores=NUM_CORES, num_subcores=NUM_SUBCORES)
    deg = pl.kernel(
        _deg_body,
        mesh=mesh,
        out_type=jax.ShapeDtypeStruct((NUM_CORES, N_PAD), jnp.float32),
        scratch_types=[
            pltpu.VMEM((CHUNKS_PER_TILE, CHUNK), jnp.int32),
            pltpu.VMEM((CHUNK,), jnp.float32),
            pltpu.VMEM_SHARED((N_PAD,), jnp.float32),
            pltpu.SemaphoreType.DMA((DEG_SEMS,)),
        ],
    )
    scatter = pl.kernel(
        _scatter_body,
        mesh=mesh,
        out_type=jax.ShapeDtypeStruct((NUM_CORES, N_PAD, D), jnp.float32),
        scratch_types=[
            pltpu.VMEM((NBUF, CHUNK, D), jnp.float32),
            pltpu.VMEM((NIG, 2 * GROUP, CHUNK), jnp.int32),
            pltpu.VMEM_SHARED((N_PAD, D), jnp.float32),
            pltpu.SemaphoreType.DMA((NBUF,)),
            pltpu.SemaphoreType.DMA((NIG,)),
            pltpu.SemaphoreType.DMA((NBUF,)),
            pltpu.SemaphoreType.DMA,
        ],
    )
    return deg, scatter


@jax.jit
def kernel(x, edge_index, W1, b1, W2, b2):
    pad = E_PAD - E
    ar = jnp.arange(pad, dtype=jnp.int32)
    src_p = jnp.concatenate([edge_index[0], ar % N])
    dst_p = jnp.concatenate([edge_index[1], N + (ar % (N_PAD - N))])
    src_p = src_p.reshape(NUM_WORKERS, CHUNKS_PER_TILE, CHUNK)
    dst_p = dst_p.reshape(NUM_WORKERS, CHUNKS_PER_TILE, CHUNK)
    edges = jnp.stack([src_p, dst_p], axis=2).reshape(
        NUM_WORKERS, NUM_GROUPS, 2 * GROUP, CHUNK)

    b1r = b1.reshape(1, D)
    b2r = b2.reshape(1, D)

    deg_kernel, scatter_kernel = _sc_kernels()
    deg_t = deg_kernel(dst_p).T
    g1 = _tc1(deg_t, x, W1)
    s1 = scatter_kernel(g1, edges)
    g2 = _tc2(deg_t, s1, W2, b1r)
    s2 = scatter_kernel(g2, edges)
    return _tc3(deg_t, s2, b2r)
